# baseline (speedup 1.0000x reference)
import functools

import jax
import jax.numpy as jnp
from jax import lax
from jax.experimental import pallas as pl
from jax.experimental.pallas import tpu as pltpu
from jax.experimental.pallas import tpu_sc as plsc

S = 2048
D = 768
E = 8
K = 2
H = 3072
CAP = 512
CAPB = 520
NEG = -1e30

NC, NS = 2, 16
NW = NC * NS
TPW = S // NW
CH = 32


def _router_body(x_ref, noise_ref, wg_ref, bg_ref, wn_ref, bn_ref,
                 ws_ref, bs_ref, d0_ref, d1_ref, g0_ref, g1_ref, xres_ref):
    x = x_ref[...]
    logits = jnp.dot(x, wg_ref[...], preferred_element_type=jnp.float32) + bg_ref[...]
    nlog = jnp.dot(x, wn_ref[...], preferred_element_type=jnp.float32) + bn_ref[...]
    sp = jnp.maximum(nlog, 0.0) + jnp.log(1.0 + jnp.exp(-jnp.abs(nlog)))
    noisy = logits + noise_ref[...] * sp

    iota_e = jax.lax.broadcasted_iota(jnp.int32, (S, E), 1)
    m1 = jnp.max(noisy, axis=1, keepdims=True)
    e1 = jnp.min(jnp.where(noisy == m1, iota_e, E), axis=1, keepdims=True)
    masked = jnp.where(iota_e == e1, NEG, noisy)
    m2 = jnp.max(masked, axis=1, keepdims=True)
    e2 = jnp.min(jnp.where(masked == m2, iota_e, E), axis=1, keepdims=True)
    sel = (iota_e == e1) | (iota_e == e2)

    ex = jnp.where(sel, jnp.exp(noisy - m1), 0.0)
    gate = ex / jnp.sum(ex, axis=1, keepdims=True)

    slogit = jnp.dot(x, ws_ref[...], preferred_element_type=jnp.float32) + bs_ref[...]
    ns = (slogit <= 0.0).astype(jnp.float32)
    n_ns = jnp.sum(ns)
    cap = jnp.floor(n_ns * (K / E))

    m = jnp.where(sel, ns, 0.0)
    ci = jax.lax.broadcasted_iota(jnp.int32, (128, 128), 0)
    cj = jax.lax.broadcasted_iota(jnp.int32, (128, 128), 1)
    ltri = (ci >= cj).astype(jnp.float32)
    off = jnp.zeros((1, E), jnp.float32)
    ranks = []
    for c in range(S // 128):
        mc = m[c * 128:(c + 1) * 128, :]
        incl = jnp.dot(ltri, mc, preferred_element_type=jnp.float32) + off
        ranks.append(incl - 1.0)
        off = off + jnp.sum(mc, axis=0, keepdims=True)
    rank = jnp.concatenate(ranks, axis=0)
    keepf = jnp.where((m > 0.0) & (rank < cap), 1.0, 0.0)

    h1 = (iota_e == e1)
    h2 = (iota_e == e2)
    r1 = jnp.sum(jnp.where(h1, rank, 0.0), axis=1, keepdims=True)
    r2 = jnp.sum(jnp.where(h2, rank, 0.0), axis=1, keepdims=True)
    k1 = jnp.sum(jnp.where(h1, keepf, 0.0), axis=1, keepdims=True)
    k2 = jnp.sum(jnp.where(h2, keepf, 0.0), axis=1, keepdims=True)
    g1v = jnp.sum(jnp.where(h1, gate * keepf, 0.0), axis=1, keepdims=True)
    g2v = jnp.sum(jnp.where(h2, gate * keepf, 0.0), axis=1, keepdims=True)

    e1f = e1.astype(jnp.float32)
    e2f = e2.astype(jnp.float32)
    d0_ref[...] = (e1f * CAPB + jnp.where(k1 > 0.0, r1, CAP)).astype(jnp.int32)
    d1_ref[...] = (e2f * CAPB + jnp.where(k2 > 0.0, r2, CAP)).astype(jnp.int32)
    ones16 = jnp.ones((1, 128), jnp.float32)
    g0_ref[...] = g1v * ones16
    g1_ref[...] = g2v * ones16
    xres_ref[...] = jnp.where(ns > 0.0, 0.0, x)


@functools.lru_cache(maxsize=None)
def _make_sc_dispatch():
    mesh = plsc.VectorSubcoreMesh(core_axis_name="c", subcore_axis_name="s")
    return functools.partial(
        pl.kernel, mesh=mesh,
        out_type=(
            jax.ShapeDtypeStruct((E * CAPB, D), jnp.float32),
            jax.ShapeDtypeStruct((E * CAPB, 128), jnp.float32),
        ),
        scratch_types=[
            pltpu.VMEM((TPW,), jnp.int32),
            pltpu.VMEM((TPW,), jnp.int32),
            pltpu.VMEM((TPW, D), jnp.float32),
            pltpu.VMEM((TPW, 128), jnp.float32),
            pltpu.VMEM((TPW, 128), jnp.float32),
            pltpu.SemaphoreType.DMA,
        ],
    )(_sc_dispatch_body)


def _sc_dispatch_body(x_hbm, d0_hbm, d1_hbm, g0_hbm, g1_hbm,
                      xd_hbm, gd_hbm, i0_v, i1_v, xr_v, g0_v, g1_v, sem):
    wid = lax.axis_index("s") * NC + lax.axis_index("c")
    base = wid * TPW
    pltpu.sync_copy(d0_hbm.at[pl.ds(base, TPW)], i0_v)
    pltpu.sync_copy(d1_hbm.at[pl.ds(base, TPW)], i1_v)
    pltpu.sync_copy(x_hbm.at[pl.ds(base, TPW)], xr_v)
    pltpu.sync_copy(g0_hbm.at[pl.ds(base, TPW)], g0_v)
    pltpu.sync_copy(g1_hbm.at[pl.ds(base, TPW)], g1_v)
    pltpu.async_copy(xr_v, xd_hbm.at[i0_v], sem).wait()
    pltpu.async_copy(xr_v, xd_hbm.at[i1_v], sem).wait()
    pltpu.async_copy(g0_v, gd_hbm.at[i0_v], sem).wait()
    pltpu.async_copy(g1_v, gd_hbm.at[i1_v], sem).wait()


def _mlp_body(nhc, xd_ref, gd_ref, w1_ref, b1_ref, w2_ref, b2_ref,
              yo_ref, y_scr):
    hc = pl.program_id(1)
    xe = xd_ref[...]
    xs = jnp.where(jnp.abs(xe) < 1e30, xe, 0.0).astype(jnp.bfloat16)
    h = jnp.maximum(
        jnp.dot(xs, w1_ref[0], preferred_element_type=jnp.float32)
        + b1_ref[0], 0.0).astype(jnp.bfloat16)
    part = jnp.dot(h, w2_ref[0], preferred_element_type=jnp.float32)

    @pl.when(hc == 0)
    def _y0():
        y_scr[...] = part

    @pl.when(hc != 0)
    def _yn():
        y_scr[...] = y_scr[...] + part

    @pl.when(hc == nhc - 1)
    def _emit():
        g = gd_ref[...][:, 0:1]
        gc = jnp.where(jnp.abs(g) <= 1.0, g, 0.0)
        yo_ref[...] = (y_scr[...] + b2_ref[0]) * gc


@functools.lru_cache(maxsize=None)
def _make_sc_combine():
    mesh = plsc.VectorSubcoreMesh(core_axis_name="c", subcore_axis_name="s")
    return functools.partial(
        pl.kernel, mesh=mesh,
        out_type=jax.ShapeDtypeStruct((S, D), jnp.float32),
        scratch_types=[
            pltpu.VMEM((CH,), jnp.int32),
            pltpu.VMEM((CH,), jnp.int32),
            pltpu.VMEM((CH, D), jnp.float32),
            pltpu.VMEM((CH, D), jnp.float32),
            pltpu.VMEM((CH, D), jnp.float32),
            pltpu.SemaphoreType.DMA,
        ],
    )(_sc_combine_body)


def _sc_combine_body(d0_hbm, d1_hbm, yo_hbm, xres_hbm, out_hbm,
                     i0_v, i1_v, y0_v, y1_v, xr_v, sem):
    wid = lax.axis_index("s") * NC + lax.axis_index("c")
    for c in range(TPW // CH):
        b = wid * TPW + c * CH
        pltpu.sync_copy(d0_hbm.at[pl.ds(b, CH)], i0_v)
        pltpu.sync_copy(d1_hbm.at[pl.ds(b, CH)], i1_v)
        pltpu.async_copy(yo_hbm.at[i0_v], y0_v, sem).wait()
        pltpu.async_copy(yo_hbm.at[i1_v], y1_v, sem).wait()
        pltpu.sync_copy(xres_hbm.at[pl.ds(b, CH)], xr_v)

        def _tb(t, carry):
            def _jb(j, carry2):
                sl = pl.ds(j * 16, 16)
                xr_v[t, sl] = xr_v[t, sl] + y0_v[t, sl] + y1_v[t, sl]
                return carry2
            return jax.lax.fori_loop(0, D // 16, _jb, carry)

        jax.lax.fori_loop(0, CH, _tb, 0)
        pltpu.sync_copy(xr_v, out_hbm.at[pl.ds(b, CH)])


def kernel(x, noise, Wg, bg, Wn, bn, Ws, bs, W1, b1, W2, b2):
    xf = x.reshape(S, D)
    nf = noise.reshape(S, E)

    d0, d1, g0m, g1m, xres = pl.pallas_call(
        _router_body,
        out_shape=(
            jax.ShapeDtypeStruct((S, 1), jnp.int32),
            jax.ShapeDtypeStruct((S, 1), jnp.int32),
            jax.ShapeDtypeStruct((S, 128), jnp.float32),
            jax.ShapeDtypeStruct((S, 128), jnp.float32),
            jax.ShapeDtypeStruct((S, D), jnp.float32),
        ),
    )(xf, nf, Wg, bg.reshape(1, E), Wn, bn.reshape(1, E),
      Ws, bs.reshape(1, 1))

    xd, gd = _make_sc_dispatch()(xf, d0.reshape(S), d1.reshape(S), g0m, g1m)

    hblk = 1536
    nhc = H // hblk
    yo = pl.pallas_call(
        functools.partial(_mlp_body, nhc),
        grid=(E, nhc),
        in_specs=[
            pl.BlockSpec((CAPB, D), lambda e, hc: (e, 0)),
            pl.BlockSpec((CAPB, 128), lambda e, hc: (e, 0)),
            pl.BlockSpec((1, D, hblk), lambda e, hc: (e, 0, hc)),
            pl.BlockSpec((1, 1, hblk), lambda e, hc: (e, 0, hc)),
            pl.BlockSpec((1, hblk, D), lambda e, hc: (e, hc, 0)),
            pl.BlockSpec((1, 1, D), lambda e, hc: (e, 0, 0)),
        ],
        out_specs=pl.BlockSpec((CAPB, D), lambda e, hc: (e, 0)),
        out_shape=jax.ShapeDtypeStruct((E * CAPB, D), jnp.float32),
        scratch_shapes=[pltpu.VMEM((CAPB, D), jnp.float32)],
    )(xd, gd, W1.astype(jnp.bfloat16), b1.reshape(E, 1, H),
      W2.astype(jnp.bfloat16), b2.reshape(E, 1, D))

    out = _make_sc_combine()(d0.reshape(S), d1.reshape(S), yo, xres)
    return out.reshape(1, S, D)


# speedup vs baseline: 1.0615x; 1.0615x over previous
import functools

import jax
import jax.numpy as jnp
from jax import lax
from jax.experimental import pallas as pl
from jax.experimental.pallas import tpu as pltpu
from jax.experimental.pallas import tpu_sc as plsc

S = 2048
D = 768
E = 8
K = 2
H = 3072
CAP = 512
CAPB = 520
NEG = -1e30

NC, NS = 2, 16
NW = NC * NS
TPW = S // NW
CH = 32


def _router_body(x_ref, noise_ref, wg_ref, bg_ref, wn_ref, bn_ref,
                 ws_ref, bs_ref, d0_ref, d1_ref, g0_ref, g1_ref, xres_ref):
    x = x_ref[...]
    logits = jnp.dot(x, wg_ref[...], preferred_element_type=jnp.float32) + bg_ref[...]
    nlog = jnp.dot(x, wn_ref[...], preferred_element_type=jnp.float32) + bn_ref[...]
    sp = jnp.maximum(nlog, 0.0) + jnp.log(1.0 + jnp.exp(-jnp.abs(nlog)))
    noisy = logits + noise_ref[...] * sp

    iota_e = jax.lax.broadcasted_iota(jnp.int32, (S, E), 1)
    m1 = jnp.max(noisy, axis=1, keepdims=True)
    e1 = jnp.min(jnp.where(noisy == m1, iota_e, E), axis=1, keepdims=True)
    masked = jnp.where(iota_e == e1, NEG, noisy)
    m2 = jnp.max(masked, axis=1, keepdims=True)
    e2 = jnp.min(jnp.where(masked == m2, iota_e, E), axis=1, keepdims=True)
    sel = (iota_e == e1) | (iota_e == e2)

    ex = jnp.where(sel, jnp.exp(noisy - m1), 0.0)
    gate = ex / jnp.sum(ex, axis=1, keepdims=True)

    slogit = jnp.dot(x, ws_ref[...], preferred_element_type=jnp.float32) + bs_ref[...]
    ns = (slogit <= 0.0).astype(jnp.float32)
    n_ns = jnp.sum(ns)
    cap = jnp.floor(n_ns * (K / E))

    m = jnp.where(sel, ns, 0.0)
    ci = jax.lax.broadcasted_iota(jnp.int32, (128, 128), 0)
    cj = jax.lax.broadcasted_iota(jnp.int32, (128, 128), 1)
    ltri = (ci >= cj).astype(jnp.float32)
    off = jnp.zeros((1, E), jnp.float32)
    ranks = []
    for c in range(S // 128):
        mc = m[c * 128:(c + 1) * 128, :]
        incl = jnp.dot(ltri, mc, preferred_element_type=jnp.float32) + off
        ranks.append(incl - 1.0)
        off = off + jnp.sum(mc, axis=0, keepdims=True)
    rank = jnp.concatenate(ranks, axis=0)
    keepf = jnp.where((m > 0.0) & (rank < cap), 1.0, 0.0)

    h1 = (iota_e == e1)
    h2 = (iota_e == e2)
    r1 = jnp.sum(jnp.where(h1, rank, 0.0), axis=1, keepdims=True)
    r2 = jnp.sum(jnp.where(h2, rank, 0.0), axis=1, keepdims=True)
    k1 = jnp.sum(jnp.where(h1, keepf, 0.0), axis=1, keepdims=True)
    k2 = jnp.sum(jnp.where(h2, keepf, 0.0), axis=1, keepdims=True)
    g1v = jnp.sum(jnp.where(h1, gate * keepf, 0.0), axis=1, keepdims=True)
    g2v = jnp.sum(jnp.where(h2, gate * keepf, 0.0), axis=1, keepdims=True)

    e1f = e1.astype(jnp.float32)
    e2f = e2.astype(jnp.float32)
    d0_ref[...] = (e1f * CAPB + jnp.where(k1 > 0.0, r1, CAP)).astype(jnp.int32)
    d1_ref[...] = (e2f * CAPB + jnp.where(k2 > 0.0, r2, CAP)).astype(jnp.int32)
    ones16 = jnp.ones((1, 128), jnp.float32)
    g0_ref[...] = g1v * ones16
    g1_ref[...] = g2v * ones16
    xres_ref[...] = jnp.where(ns > 0.0, 0.0, x)


@functools.lru_cache(maxsize=None)
def _make_sc_dispatch():
    mesh = plsc.VectorSubcoreMesh(core_axis_name="c", subcore_axis_name="s")
    return functools.partial(
        pl.kernel, mesh=mesh,
        out_type=(
            jax.ShapeDtypeStruct((E * CAPB, D), jnp.float32),
            jax.ShapeDtypeStruct((E * CAPB, 128), jnp.float32),
        ),
        scratch_types=[
            pltpu.VMEM((TPW,), jnp.int32),
            pltpu.VMEM((TPW,), jnp.int32),
            pltpu.VMEM((TPW, D), jnp.float32),
            pltpu.VMEM((TPW, 128), jnp.float32),
            pltpu.VMEM((TPW, 128), jnp.float32),
            pltpu.SemaphoreType.DMA,
        ],
    )(_sc_dispatch_body)


def _sc_dispatch_body(x_hbm, d0_hbm, d1_hbm, g0_hbm, g1_hbm,
                      xd_hbm, gd_hbm, i0_v, i1_v, xr_v, g0_v, g1_v, sem):
    wid = lax.axis_index("s") * NC + lax.axis_index("c")
    base = wid * TPW
    pltpu.sync_copy(d0_hbm.at[pl.ds(base, TPW)], i0_v)
    pltpu.sync_copy(d1_hbm.at[pl.ds(base, TPW)], i1_v)
    pltpu.sync_copy(x_hbm.at[pl.ds(base, TPW)], xr_v)
    pltpu.sync_copy(g0_hbm.at[pl.ds(base, TPW)], g0_v)
    pltpu.sync_copy(g1_hbm.at[pl.ds(base, TPW)], g1_v)
    c0 = pltpu.async_copy(xr_v, xd_hbm.at[i0_v], sem)
    c1 = pltpu.async_copy(xr_v, xd_hbm.at[i1_v], sem)
    c2 = pltpu.async_copy(g0_v, gd_hbm.at[i0_v], sem)
    c3 = pltpu.async_copy(g1_v, gd_hbm.at[i1_v], sem)
    c0.wait()
    c1.wait()
    c2.wait()
    c3.wait()


def _mlp_body(nhc, xd_ref, gd_ref, w1_ref, b1_ref, w2_ref, b2_ref,
              yo_ref, y_scr):
    hc = pl.program_id(1)
    xe = xd_ref[...]
    xs = jnp.where(jnp.abs(xe) < 1e30, xe, 0.0).astype(jnp.bfloat16)
    h = jnp.maximum(
        jnp.dot(xs, w1_ref[0], preferred_element_type=jnp.float32)
        + b1_ref[0], 0.0).astype(jnp.bfloat16)
    part = jnp.dot(h, w2_ref[0], preferred_element_type=jnp.float32)

    @pl.when(hc == 0)
    def _y0():
        y_scr[...] = part

    @pl.when(hc != 0)
    def _yn():
        y_scr[...] = y_scr[...] + part

    @pl.when(hc == nhc - 1)
    def _emit():
        g = gd_ref[...][:, 0:1]
        gc = jnp.where(jnp.abs(g) <= 1.0, g, 0.0)
        yo_ref[...] = (y_scr[...] + b2_ref[0]) * gc


@functools.lru_cache(maxsize=None)
def _make_sc_combine():
    mesh = plsc.VectorSubcoreMesh(core_axis_name="c", subcore_axis_name="s")
    return functools.partial(
        pl.kernel, mesh=mesh,
        out_type=jax.ShapeDtypeStruct((S, D), jnp.float32),
        scratch_types=[
            pltpu.VMEM((CH,), jnp.int32),
            pltpu.VMEM((CH,), jnp.int32),
            pltpu.VMEM((CH, D), jnp.float32),
            pltpu.VMEM((CH, D), jnp.float32),
            pltpu.VMEM((CH, D), jnp.float32),
            pltpu.SemaphoreType.DMA,
        ],
    )(_sc_combine_body)


def _sc_combine_body(d0_hbm, d1_hbm, yo_hbm, xres_hbm, out_hbm,
                     i0_v, i1_v, y0_v, y1_v, xr_v, sem):
    wid = lax.axis_index("s") * NC + lax.axis_index("c")
    for c in range(TPW // CH):
        b = wid * TPW + c * CH
        pltpu.sync_copy(d0_hbm.at[pl.ds(b, CH)], i0_v)
        pltpu.sync_copy(d1_hbm.at[pl.ds(b, CH)], i1_v)
        c0 = pltpu.async_copy(yo_hbm.at[i0_v], y0_v, sem)
        c1 = pltpu.async_copy(yo_hbm.at[i1_v], y1_v, sem)
        pltpu.sync_copy(xres_hbm.at[pl.ds(b, CH)], xr_v)
        c0.wait()
        c1.wait()

        def _tb(t, carry):
            for j in range(D // 16):
                sl = pl.ds(j * 16, 16)
                xr_v[t, sl] = xr_v[t, sl] + y0_v[t, sl] + y1_v[t, sl]
            return carry

        jax.lax.fori_loop(0, CH, _tb, 0)
        pltpu.sync_copy(xr_v, out_hbm.at[pl.ds(b, CH)])


def kernel(x, noise, Wg, bg, Wn, bn, Ws, bs, W1, b1, W2, b2):
    xf = x.reshape(S, D)
    nf = noise.reshape(S, E)

    d0, d1, g0m, g1m, xres = pl.pallas_call(
        _router_body,
        out_shape=(
            jax.ShapeDtypeStruct((S, 1), jnp.int32),
            jax.ShapeDtypeStruct((S, 1), jnp.int32),
            jax.ShapeDtypeStruct((S, 128), jnp.float32),
            jax.ShapeDtypeStruct((S, 128), jnp.float32),
            jax.ShapeDtypeStruct((S, D), jnp.float32),
        ),
    )(xf, nf, Wg, bg.reshape(1, E), Wn, bn.reshape(1, E),
      Ws, bs.reshape(1, 1))

    xd, gd = _make_sc_dispatch()(xf, d0.reshape(S), d1.reshape(S), g0m, g1m)

    hblk = 1536
    nhc = H // hblk
    yo = pl.pallas_call(
        functools.partial(_mlp_body, nhc),
        grid=(E, nhc),
        in_specs=[
            pl.BlockSpec((CAPB, D), lambda e, hc: (e, 0)),
            pl.BlockSpec((CAPB, 128), lambda e, hc: (e, 0)),
            pl.BlockSpec((1, D, hblk), lambda e, hc: (e, 0, hc)),
            pl.BlockSpec((1, 1, hblk), lambda e, hc: (e, 0, hc)),
            pl.BlockSpec((1, hblk, D), lambda e, hc: (e, hc, 0)),
            pl.BlockSpec((1, 1, D), lambda e, hc: (e, 0, 0)),
        ],
        out_specs=pl.BlockSpec((CAPB, D), lambda e, hc: (e, 0)),
        out_shape=jax.ShapeDtypeStruct((E * CAPB, D), jnp.float32),
        scratch_shapes=[pltpu.VMEM((CAPB, D), jnp.float32)],
    )(xd, gd, W1.astype(jnp.bfloat16), b1.reshape(E, 1, H),
      W2.astype(jnp.bfloat16), b2.reshape(E, 1, D))

    out = _make_sc_combine()(d0.reshape(S), d1.reshape(S), yo, xres)
    return out.reshape(1, S, D)


# speedup vs baseline: 1.0861x; 1.0232x over previous
import functools

import jax
import jax.numpy as jnp
from jax import lax
from jax.experimental import pallas as pl
from jax.experimental.pallas import tpu as pltpu
from jax.experimental.pallas import tpu_sc as plsc

S = 2048
D = 768
E = 8
K = 2
H = 3072
CAP = 512
CAPB = 520
NEG = -1e30

NC, NS = 2, 16
NW = NC * NS
TPW = S // NW
CH = 32


def _router_body(x_ref, noise_ref, wg_ref, bg_ref, wn_ref, bn_ref,
                 ws_ref, bs_ref, d0_ref, d1_ref, g0_ref, g1_ref, xres_ref):
    x = x_ref[...]
    logits = jnp.dot(x, wg_ref[...], preferred_element_type=jnp.float32) + bg_ref[...]
    nlog = jnp.dot(x, wn_ref[...], preferred_element_type=jnp.float32) + bn_ref[...]
    sp = jnp.maximum(nlog, 0.0) + jnp.log(1.0 + jnp.exp(-jnp.abs(nlog)))
    noisy = logits + noise_ref[...] * sp

    iota_e = jax.lax.broadcasted_iota(jnp.int32, (S, E), 1)
    m1 = jnp.max(noisy, axis=1, keepdims=True)
    e1 = jnp.min(jnp.where(noisy == m1, iota_e, E), axis=1, keepdims=True)
    masked = jnp.where(iota_e == e1, NEG, noisy)
    m2 = jnp.max(masked, axis=1, keepdims=True)
    e2 = jnp.min(jnp.where(masked == m2, iota_e, E), axis=1, keepdims=True)
    sel = (iota_e == e1) | (iota_e == e2)

    ex = jnp.where(sel, jnp.exp(noisy - m1), 0.0)
    gate = ex / jnp.sum(ex, axis=1, keepdims=True)

    slogit = jnp.dot(x, ws_ref[...], preferred_element_type=jnp.float32) + bs_ref[...]
    ns = (slogit <= 0.0).astype(jnp.float32)
    n_ns = jnp.sum(ns)
    cap = jnp.floor(n_ns * (K / E))

    m = jnp.where(sel, ns, 0.0)
    ci = jax.lax.broadcasted_iota(jnp.int32, (128, 128), 0)
    cj = jax.lax.broadcasted_iota(jnp.int32, (128, 128), 1)
    ltri = (ci >= cj).astype(jnp.float32)
    off = jnp.zeros((1, E), jnp.float32)
    ranks = []
    for c in range(S // 128):
        mc = m[c * 128:(c + 1) * 128, :]
        incl = jnp.dot(ltri, mc, preferred_element_type=jnp.float32) + off
        ranks.append(incl - 1.0)
        off = off + jnp.sum(mc, axis=0, keepdims=True)
    rank = jnp.concatenate(ranks, axis=0)
    keepf = jnp.where((m > 0.0) & (rank < cap), 1.0, 0.0)

    h1 = (iota_e == e1)
    h2 = (iota_e == e2)
    r1 = jnp.sum(jnp.where(h1, rank, 0.0), axis=1, keepdims=True)
    r2 = jnp.sum(jnp.where(h2, rank, 0.0), axis=1, keepdims=True)
    k1 = jnp.sum(jnp.where(h1, keepf, 0.0), axis=1, keepdims=True)
    k2 = jnp.sum(jnp.where(h2, keepf, 0.0), axis=1, keepdims=True)
    g1v = jnp.sum(jnp.where(h1, gate * keepf, 0.0), axis=1, keepdims=True)
    g2v = jnp.sum(jnp.where(h2, gate * keepf, 0.0), axis=1, keepdims=True)

    e1f = e1.astype(jnp.float32)
    e2f = e2.astype(jnp.float32)
    d0_ref[...] = (e1f * CAPB + jnp.where(k1 > 0.0, r1, CAP)).astype(jnp.int32)
    d1_ref[...] = (e2f * CAPB + jnp.where(k2 > 0.0, r2, CAP)).astype(jnp.int32)
    ones16 = jnp.ones((1, 128), jnp.float32)
    g0_ref[...] = g1v * ones16
    g1_ref[...] = g2v * ones16
    xres_ref[...] = jnp.where(ns > 0.0, 0.0, x)


@functools.lru_cache(maxsize=None)
def _make_sc_dispatch():
    mesh = plsc.VectorSubcoreMesh(core_axis_name="c", subcore_axis_name="s")
    return functools.partial(
        pl.kernel, mesh=mesh,
        out_type=(
            jax.ShapeDtypeStruct((E * CAPB, D), jnp.float32),
            jax.ShapeDtypeStruct((E * CAPB, 128), jnp.float32),
        ),
        scratch_types=[
            pltpu.VMEM((TPW,), jnp.int32),
            pltpu.VMEM((TPW,), jnp.int32),
            pltpu.VMEM((TPW, D), jnp.float32),
            pltpu.VMEM((TPW, 128), jnp.float32),
            pltpu.VMEM((TPW, 128), jnp.float32),
            pltpu.SemaphoreType.DMA,
        ],
    )(_sc_dispatch_body)


def _sc_dispatch_body(x_hbm, d0_hbm, d1_hbm, g0_hbm, g1_hbm,
                      xd_hbm, gd_hbm, i0_v, i1_v, xr_v, g0_v, g1_v, sem):
    wid = lax.axis_index("s") * NC + lax.axis_index("c")
    base = wid * TPW
    pltpu.sync_copy(d0_hbm.at[pl.ds(base, TPW)], i0_v)
    pltpu.sync_copy(d1_hbm.at[pl.ds(base, TPW)], i1_v)
    pltpu.sync_copy(x_hbm.at[pl.ds(base, TPW)], xr_v)
    pltpu.sync_copy(g0_hbm.at[pl.ds(base, TPW)], g0_v)
    pltpu.sync_copy(g1_hbm.at[pl.ds(base, TPW)], g1_v)
    c0 = pltpu.async_copy(xr_v, xd_hbm.at[i0_v], sem)
    c1 = pltpu.async_copy(xr_v, xd_hbm.at[i1_v], sem)
    c2 = pltpu.async_copy(g0_v, gd_hbm.at[i0_v], sem)
    c3 = pltpu.async_copy(g1_v, gd_hbm.at[i1_v], sem)
    c0.wait()
    c1.wait()
    c2.wait()
    c3.wait()


def _mlp_body(xd_ref, gd_ref, w1_ref, b1_ref, w2_ref, b2_ref, yo_ref):
    xe = xd_ref[...]
    xs = jnp.where(jnp.abs(xe) < 1e30, xe, 0.0).astype(jnp.bfloat16)
    h = jnp.maximum(
        jnp.dot(xs, w1_ref[0], preferred_element_type=jnp.float32)
        + b1_ref[0], 0.0).astype(jnp.bfloat16)
    part = jnp.dot(h, w2_ref[0], preferred_element_type=jnp.float32)
    g = gd_ref[...][:, 0:1]
    gc = jnp.where(jnp.abs(g) <= 1.0, g, 0.0)
    yo_ref[...] = (part + b2_ref[0]) * gc


@functools.lru_cache(maxsize=None)
def _make_sc_combine():
    mesh = plsc.VectorSubcoreMesh(core_axis_name="c", subcore_axis_name="s")
    return functools.partial(
        pl.kernel, mesh=mesh,
        out_type=jax.ShapeDtypeStruct((S, D), jnp.float32),
        scratch_types=[
            pltpu.VMEM((CH,), jnp.int32),
            pltpu.VMEM((CH,), jnp.int32),
            pltpu.VMEM((CH, D), jnp.float32),
            pltpu.VMEM((CH, D), jnp.float32),
            pltpu.VMEM((CH, D), jnp.float32),
            pltpu.SemaphoreType.DMA,
        ],
    )(_sc_combine_body)


def _sc_combine_body(d0_hbm, d1_hbm, yo_hbm, xres_hbm, out_hbm,
                     i0_v, i1_v, y0_v, y1_v, xr_v, sem):
    wid = lax.axis_index("s") * NC + lax.axis_index("c")
    for c in range(TPW // CH):
        b = wid * TPW + c * CH
        pltpu.sync_copy(d0_hbm.at[pl.ds(b, CH)], i0_v)
        pltpu.sync_copy(d1_hbm.at[pl.ds(b, CH)], i1_v)
        c0 = pltpu.async_copy(yo_hbm.at[i0_v], y0_v, sem)
        c1 = pltpu.async_copy(yo_hbm.at[i1_v], y1_v, sem)
        pltpu.sync_copy(xres_hbm.at[pl.ds(b, CH)], xr_v)
        c0.wait()
        c1.wait()

        def _tb(t, carry):
            for j in range(D // 16):
                sl = pl.ds(j * 16, 16)
                xr_v[t, sl] = xr_v[t, sl] + y0_v[t, sl] + y1_v[t, sl]
            return carry

        jax.lax.fori_loop(0, CH, _tb, 0)
        pltpu.sync_copy(xr_v, out_hbm.at[pl.ds(b, CH)])


def kernel(x, noise, Wg, bg, Wn, bn, Ws, bs, W1, b1, W2, b2):
    xf = x.reshape(S, D)
    nf = noise.reshape(S, E)

    d0, d1, g0m, g1m, xres = pl.pallas_call(
        _router_body,
        out_shape=(
            jax.ShapeDtypeStruct((S, 1), jnp.int32),
            jax.ShapeDtypeStruct((S, 1), jnp.int32),
            jax.ShapeDtypeStruct((S, 128), jnp.float32),
            jax.ShapeDtypeStruct((S, 128), jnp.float32),
            jax.ShapeDtypeStruct((S, D), jnp.float32),
        ),
    )(xf, nf, Wg, bg.reshape(1, E), Wn, bn.reshape(1, E),
      Ws, bs.reshape(1, 1))

    xd, gd = _make_sc_dispatch()(xf, d0.reshape(S), d1.reshape(S), g0m, g1m)

    yo = pl.pallas_call(
        _mlp_body,
        grid=(E,),
        in_specs=[
            pl.BlockSpec((CAPB, D), lambda e: (e, 0)),
            pl.BlockSpec((CAPB, 128), lambda e: (e, 0)),
            pl.BlockSpec((1, D, H), lambda e: (e, 0, 0)),
            pl.BlockSpec((1, 1, H), lambda e: (e, 0, 0)),
            pl.BlockSpec((1, H, D), lambda e: (e, 0, 0)),
            pl.BlockSpec((1, 1, D), lambda e: (e, 0, 0)),
        ],
        out_specs=pl.BlockSpec((CAPB, D), lambda e: (e, 0)),
        out_shape=jax.ShapeDtypeStruct((E * CAPB, D), jnp.float32),
    )(xd, gd, W1.astype(jnp.bfloat16), b1.reshape(E, 1, H),
      W2.astype(jnp.bfloat16), b2.reshape(E, 1, D))

    out = _make_sc_combine()(d0.reshape(S), d1.reshape(S), yo, xres)
    return out.reshape(1, S, D)
